# baseline (speedup 1.0000x reference)
import jax
import jax.numpy as jnp
from jax.experimental import pallas as pl
from jax.experimental.pallas import tpu as pltpu

DIM = 128


def _copy_body(x_ref, o_ref, sem):
    pltpu.make_async_copy(x_ref, o_ref, sem).start()
    pltpu.make_async_copy(x_ref, o_ref, sem).wait()


def kernel(x):
    x = x.astype(jnp.float32)
    n = x.size // DIM
    return pl.pallas_call(
        _copy_body,
        out_shape=jax.ShapeDtypeStruct((n, DIM), jnp.float32),
        in_specs=[pl.BlockSpec(memory_space=pltpu.MemorySpace.HBM)],
        out_specs=pl.BlockSpec(memory_space=pltpu.MemorySpace.HBM),
        scratch_shapes=[pltpu.SemaphoreType.DMA],
    )(x.reshape(n, DIM))


# speedup vs baseline: 29.1909x; 29.1909x over previous
import jax
import jax.numpy as jnp
from jax.experimental import pallas as pl
from jax.experimental.pallas import tpu as pltpu

DIM = 128
BLOCK_ROWS = 2048


def _copy_body(x_ref, o_ref):
    o_ref[...] = x_ref[...]


def kernel(x):
    x = x.astype(jnp.float32)
    n = x.size // DIM
    x = x.reshape(n, DIM)
    grid = (n // BLOCK_ROWS,)
    return pl.pallas_call(
        _copy_body,
        out_shape=jax.ShapeDtypeStruct((n, DIM), jnp.float32),
        grid=grid,
        in_specs=[pl.BlockSpec((BLOCK_ROWS, DIM), lambda i: (i, 0))],
        out_specs=pl.BlockSpec((BLOCK_ROWS, DIM), lambda i: (i, 0)),
    )(x)
